# baseline (speedup 1.0000x reference)
import functools

import jax
import jax.numpy as jnp
from jax import lax
from jax.experimental import pallas as pl
from jax.experimental.pallas import tpu as pltpu
from jax.experimental.pallas import tpu_sc as plsc

_N = 10000
_E = 160000
_D = 256
_NV = 64
_EV = 8
_EPS = 1e-5

_NC = 2
_NS = 16
_NW = _NC * _NS

_N_WORKERS = 25
_N_PER_W = 400
_N_CHUNKS = 4
_N_TAIL = 16
_E_PER_W = 5000
_E_CHUNKS = 40
_E_TAIL = 8
_C = 128


def _prep_body(x_ref, e_ref, ntab_ref, etab_ref, ng_ref, nb_ref,
               eg_ref, eb_ref, nout_ref, eout_ref):
    x2 = x_ref[...]
    e2 = e_ref[...]
    ntab = ntab_ref[...]
    etab = etab_ref[...]

    cn = [jnp.sum(jnp.where(x2 == t, 1.0, 0.0)) for t in range(_NV)]
    ce = [jnp.sum(jnp.where(e2 == t, 1.0, 0.0)) for t in range(_EV)]

    mean_n = sum(cn[t] * ntab[t:t + 1] for t in range(_NV)) * (1.0 / _N)
    dev_n = ntab - mean_n
    sq_n = dev_n * dev_n
    var_n = sum(cn[t] * sq_n[t:t + 1] for t in range(_NV)) * (1.0 / _N)
    scale_n = ng_ref[...] * lax.rsqrt(var_n + _EPS)
    nout_ref[...] = dev_n * scale_n + nb_ref[...]

    mean_e = sum(ce[t] * etab[t:t + 1] for t in range(_EV)) * (1.0 / _E)
    dev_e = etab - mean_e
    sq_e = dev_e * dev_e
    var_e = sum(ce[t] * sq_e[t:t + 1] for t in range(_EV)) * (1.0 / _E)
    scale_e = eg_ref[...] * lax.rsqrt(var_e + _EPS)
    eout_ref[...] = dev_e * scale_e + eb_ref[...]


_prep_call = pl.pallas_call(
    _prep_body,
    out_shape=(
        jax.ShapeDtypeStruct((_NV, _D), jnp.float32),
        jax.ShapeDtypeStruct((_EV, _D), jnp.float32),
    ),
)


def _gather_body(ntab, etab, nidx_hbm, eidx_hbm, h_out, e_out,
                 nidx_v, eidx_v, buf0, buf1, sem0, sem1):
    wid = lax.axis_index("s") * _NC + lax.axis_index("c")
    pltpu.sync_copy(nidx_hbm.at[wid], nidx_v)
    pltpu.sync_copy(eidx_hbm.at[wid], eidx_v)
    bufs = (buf0, buf1)
    sems = (sem0, sem1)

    @pl.when(wid < _N_WORKERS)
    def _node_phase():
        nbase = wid * _N_PER_W
        pltpu.async_copy(ntab.at[nidx_v.at[0]], buf0, sem0)
        for j in range(_N_CHUNKS):
            if j + 1 < _N_CHUNKS:
                pltpu.async_copy(ntab.at[nidx_v.at[j + 1]],
                                 bufs[(j + 1) % 2], sems[(j + 1) % 2])
            pltpu.make_async_copy(ntab.at[nidx_v.at[j]],
                                  bufs[j % 2], sems[j % 2]).wait()
            if j + 1 < _N_CHUNKS:
                pltpu.sync_copy(bufs[j % 2],
                                h_out.at[pl.ds(nbase + j * _C, _C)])
            else:
                pltpu.sync_copy(bufs[j % 2].at[pl.ds(0, _N_TAIL)],
                                h_out.at[pl.ds(nbase + j * _C, _N_TAIL)])

    ebase = wid * _E_PER_W
    pltpu.async_copy(etab.at[eidx_v.at[0]], buf0, sem0)

    def _pair(g, carry):
        a = 2 * g
        pltpu.async_copy(etab.at[eidx_v.at[a + 1]], buf1, sem1)
        pltpu.make_async_copy(etab.at[eidx_v.at[a]], buf0, sem0).wait()
        pltpu.sync_copy(buf0, e_out.at[pl.ds(ebase + a * _C, _C)])
        pltpu.async_copy(etab.at[eidx_v.at[a + 2]], buf0, sem0)
        pltpu.make_async_copy(etab.at[eidx_v.at[a + 1]], buf1, sem1).wait()
        pltpu.sync_copy(buf1, e_out.at[pl.ds(ebase + (a + 1) * _C, _C)])
        return carry

    lax.fori_loop(0, (_E_CHUNKS - 2) // 2, _pair, 0)
    a = _E_CHUNKS - 2
    pltpu.async_copy(etab.at[eidx_v.at[a + 1]], buf1, sem1)
    pltpu.make_async_copy(etab.at[eidx_v.at[a]], buf0, sem0).wait()
    pltpu.sync_copy(buf0, e_out.at[pl.ds(ebase + a * _C, _C)])
    pltpu.make_async_copy(etab.at[eidx_v.at[a + 1]], buf1, sem1).wait()
    pltpu.sync_copy(buf1.at[pl.ds(0, _E_TAIL)],
                    e_out.at[pl.ds(ebase + (a + 1) * _C, _E_TAIL)])


_gather_call = functools.partial(
    pl.kernel,
    mesh=plsc.VectorSubcoreMesh(core_axis_name="c", subcore_axis_name="s"),
    out_type=(
        jax.ShapeDtypeStruct((_N, _D), jnp.float32),
        jax.ShapeDtypeStruct((_E, _D), jnp.float32),
    ),
    scratch_types=[
        pltpu.VMEM((_N_CHUNKS, _C), jnp.int32),
        pltpu.VMEM((_E_CHUNKS, _C), jnp.int32),
        pltpu.VMEM((_C, _D), jnp.float32),
        pltpu.VMEM((_C, _D), jnp.float32),
        pltpu.SemaphoreType.DMA,
        pltpu.SemaphoreType.DMA,
    ],
)(_gather_body)


def kernel(x, edge_index, edge_attr, node_table, edge_table,
           node_bn_gamma, node_bn_beta, edge_bn_gamma, edge_bn_beta):
    x_pad = jnp.full((80 * 128,), 127, jnp.int32).at[:_N].set(x)
    ntab_n, etab_n = _prep_call(
        x_pad.reshape(80, 128),
        edge_attr.reshape(1250, 128),
        node_table, edge_table,
        node_bn_gamma.reshape(1, _D), node_bn_beta.reshape(1, _D),
        edge_bn_gamma.reshape(1, _D), edge_bn_beta.reshape(1, _D),
    )

    nidx = jnp.zeros((_NW, _N_CHUNKS * _C), jnp.int32)
    nidx = nidx.at[:_N_WORKERS, :_N_PER_W].set(x.reshape(_N_WORKERS, _N_PER_W))
    nidx = nidx.reshape(_NW, _N_CHUNKS, _C)
    eidx = jnp.zeros((_NW, _E_CHUNKS * _C), jnp.int32)
    eidx = eidx.at[:, :_E_PER_W].set(edge_attr.reshape(_NW, _E_PER_W))
    eidx = eidx.reshape(_NW, _E_CHUNKS, _C)

    h, e = _gather_call(ntab_n, etab_n, nidx, eidx)
    return (h, e)
